# baseline (speedup 1.0000x reference)
import functools

import jax
import jax.numpy as jnp
from jax import lax
from jax.experimental import pallas as pl
from jax.experimental.pallas import tpu as pltpu
from jax.experimental.pallas import tpu_sc as plsc

L = 16
NC, NS = 2, 16
NW = NC * NS


def kernel(objs, vectors, lengths):
    B, = objs.shape
    N, P, D = vectors.shape
    row = P * D
    vec2d = vectors.reshape(N, row)

    bpw = B // NW
    C = 64
    nch = bpw // C
    LCH = 128

    mesh = plsc.VectorSubcoreMesh(core_axis_name="c", subcore_axis_name="s",
                                  num_cores=NC, num_subcores=NS)

    @functools.partial(
        pl.kernel,
        out_type=jax.ShapeDtypeStruct((B, D), jnp.float32),
        mesh=mesh,
        compiler_params=pltpu.CompilerParams(use_tc_tiling_on_sc=False),
        scratch_types=[
            pltpu.VMEM((bpw,), jnp.int32),
            pltpu.VMEM((bpw,), jnp.int32),
            pltpu.VMEM((bpw + L,), jnp.float32),
            pltpu.VMEM((C, row), jnp.float32),
            pltpu.VMEM((C, row), jnp.float32),
            pltpu.VMEM((bpw, D), jnp.float32),
            pltpu.SemaphoreType.DMA,
            pltpu.SemaphoreType.DMA,
            pltpu.SemaphoreType.DMA,
        ],
    )
    def sc_kernel(objs_hbm, vec_hbm, len_hbm, out_hbm,
                  idx_v, lens_v, recip_v, buf0, buf1, out_v,
                  lsem, sem0, sem1):
        wid = lax.axis_index("s") * NC + lax.axis_index("c")
        base = wid * bpw
        bufs = (buf0, buf1)
        sems = (sem0, sem1)

        pltpu.sync_copy(objs_hbm.at[pl.ds(base, bpw)], idx_v)

        lens_handles = [
            pltpu.async_copy(len_hbm.at[idx_v.at[pl.ds(k * LCH, LCH)]],
                             lens_v.at[pl.ds(k * LCH, LCH)], lsem)
            for k in range(bpw // LCH)
        ]

        handles = {}

        def start(c):
            b = c % 2
            handles[c] = pltpu.async_copy(
                vec_hbm.at[idx_v.at[pl.ds(c * C, C)]], bufs[b], sems[b])

        start(0)

        for h in lens_handles:
            h.wait()
        for g in range(bpw // L):
            lv = lens_v[pl.ds(g * L, L)]
            recip_v[pl.ds(g * L, L)] = 1.0 / (lv.astype(jnp.float32) + 1e-5)

        zero_idx = jnp.zeros((L, 1), jnp.int32)
        bcast_dnums = lax.GatherDimensionNumbers(
            offset_dims=(), collapsed_slice_dims=(0,), start_index_map=(0,))

        def bcast0(v):
            return lax.gather(v, zero_idx, bcast_dnums, (1,),
                              mode=lax.GatherScatterMode.PROMISE_IN_BOUNDS)

        for c in range(nch):
            if c + 1 < nch:
                start(c + 1)
            handles[c].wait()
            buf = bufs[c % 2]

            def body(i, _, buf=buf, c=c):
                j = c * C + i
                rv = recip_v[pl.ds(j, L)]
                rcp = bcast0(rv)
                for db in range(D // L):
                    acc = buf[i, pl.ds(db * L, L)]
                    for p in range(1, P):
                        acc = acc + buf[i, pl.ds(p * D + db * L, L)]
                    out_v[j, pl.ds(db * L, L)] = acc * rcp
                return 0

            lax.fori_loop(0, C, body, 0)

        pltpu.sync_copy(out_v, out_hbm.at[pl.ds(base, bpw)])

    return sc_kernel(objs, vec2d, lengths)
